# Initial kernel scaffold; baseline (speedup 1.0000x reference)
import jax
import jax.numpy as jnp
from jax.experimental import pallas as pl


def kernel(xyz, conv1_w, conv1_b, conv2_w, conv2_b, bn_w, bn_b):
    raise NotImplementedError("write your pallas kernel here")


# speedup vs baseline: 13.0783x; 13.0783x over previous
import functools

import jax
import jax.numpy as jnp
from jax import lax
from jax.experimental import pallas as pl
from jax.experimental.pallas import tpu as pltpu
from jax.experimental.pallas import tpu_sc as plsc

_K = 16
_D = 16
_TR = 256
_NM = 512
_NC = 512
_CH = 128


def _knn_body(xyz_full_ref, xyz_tile_ref, idx_ref):
    b = pl.program_id(0)
    n = xyz_full_ref.shape[2]
    tr = xyz_tile_ref.shape[2]
    x = xyz_full_ref[0]
    xt = xyz_tile_ref[0]
    sq = jnp.sum(x * x, axis=0, keepdims=True)
    sqt = jnp.sum(xt * xt, axis=0)[:, None]
    cross = lax.dot_general(xt, x, (((0,), (0,)), ((), ())),
                            preferred_element_type=jnp.float32)
    d = (sqt + sq) - 2.0 * cross
    iota = lax.broadcasted_iota(jnp.int32, (tr, n), 1)
    inf = jnp.float32(jnp.inf)
    cols = []
    for k in range(_K + 1):
        m = jnp.min(d, axis=1, keepdims=True)
        eq = d == m
        sel = jnp.min(jnp.where(eq, iota, jnp.int32(n)), axis=1,
                      keepdims=True)
        if k > 0:
            cols.append(sel)
        d = jnp.where(iota == sel, inf, d)
    idxs = jnp.concatenate(cols, axis=1)
    idx_ref[0] = idxs + b * n


def _knn_indices(xyz):
    b, _, n = xyz.shape
    grid = (b, n // _TR)
    return pl.pallas_call(
        _knn_body,
        grid=grid,
        in_specs=[
            pl.BlockSpec((1, 3, n), lambda i, j: (i, 0, 0)),
            pl.BlockSpec((1, 3, _TR), lambda i, j: (i, 0, j)),
        ],
        out_specs=pl.BlockSpec((1, _TR, _K), lambda i, j: (i, j, 0)),
        out_shape=jax.ShapeDtypeStruct((b, n, _K), jnp.int32),
    )(xyz, xyz)


def _gather_rows(table, idx2d):
    rows_total = idx2d.shape[0] * idx2d.shape[1]
    info = plsc.get_sparse_core_info()
    nw = info.num_cores * info.num_subcores
    per_w = rows_total // nw
    n_ch = per_w // _CH

    mesh = plsc.VectorSubcoreMesh(core_axis_name="c", subcore_axis_name="s")

    @functools.partial(
        pl.kernel,
        mesh=mesh,
        compiler_params=pltpu.CompilerParams(use_tc_tiling_on_sc=False),
        out_type=jax.ShapeDtypeStruct((rows_total, _D), jnp.float32),
        scratch_types=[
            pltpu.VMEM((n_ch, _CH), jnp.int32),
            pltpu.VMEM((_CH, _D), jnp.float32),
            pltpu.SemaphoreType.DMA,
        ],
    )
    def _gather_kernel(table_hbm, idx_hbm, out_hbm, idx_v, rows_v, sem):
        wid = lax.axis_index("s") * info.num_cores + lax.axis_index("c")
        base = wid * per_w
        pltpu.sync_copy(idx_hbm.at[pl.ds(wid * n_ch, n_ch)], idx_v)

        def body(j, carry):
            pltpu.async_copy(table_hbm.at[idx_v.at[j]], rows_v, sem).wait()
            pltpu.sync_copy(rows_v, out_hbm.at[pl.ds(base + j * _CH, _CH)])
            return carry

        lax.fori_loop(0, n_ch, body, 0)

    return _gather_kernel(table, idx2d)


def _moments_body(g_ref, c_ref, s_ref, v_ref):
    first = (pl.program_id(0) == 0) & (pl.program_id(1) == 0)
    g = g_ref[0]
    c = c_ref[0]
    diffs = g - c[:, None, :]
    xm = diffs.reshape(g.shape[0] * _K, _D)
    smat = lax.dot_general(xm, xm, (((0,), (0,)), ((), ())),
                           preferred_element_type=jnp.float32)
    vrow = jnp.sum(xm, axis=0, keepdims=True)

    @pl.when(first)
    def _():
        s_ref[...] = jnp.zeros_like(s_ref)
        v_ref[...] = jnp.zeros_like(v_ref)

    s_ref[...] += smat
    v_ref[0:1, :] += vrow


def _moments(gathered4, centers3):
    b, n = centers3.shape[0], centers3.shape[1]
    grid = (b, n // _NM)
    return pl.pallas_call(
        _moments_body,
        grid=grid,
        in_specs=[
            pl.BlockSpec((1, _NM, _K, _D), lambda i, j: (i, j, 0, 0)),
            pl.BlockSpec((1, _NM, _D), lambda i, j: (i, j, 0)),
        ],
        out_specs=[
            pl.BlockSpec((_D, _D), lambda i, j: (0, 0)),
            pl.BlockSpec((8, _D), lambda i, j: (0, 0)),
        ],
        out_shape=[
            jax.ShapeDtypeStruct((_D, _D), jnp.float32),
            jax.ShapeDtypeStruct((8, _D), jnp.float32),
        ],
    )(gathered4, centers3)


def _mlp_body(g_ref, c_ref, w1_ref, b1_ref, w2_ref, b2_ref, o_ref):
    g = g_ref[0]
    c = c_ref[0]
    diffs = g - c[:, None, :]
    xm = diffs.reshape(g.shape[0] * _K, _D)
    h = jnp.dot(xm, w1_ref[...], preferred_element_type=jnp.float32)
    h = jnp.maximum(h + b1_ref[0:1, :], 0.0)
    z = jnp.dot(h, w2_ref[...], preferred_element_type=jnp.float32)
    z = z + b2_ref[0:1, :]
    z3 = z.reshape(g.shape[0], _K, z.shape[1])
    o = jnp.max(z3, axis=1)
    o_ref[0] = jnp.transpose(o)


def _mlp(gathered4, centers3, w1t, b1, w2t, b2):
    b, n = centers3.shape[0], centers3.shape[1]
    cout = w2t.shape[1]
    grid = (b, n // _NC)
    return pl.pallas_call(
        _mlp_body,
        grid=grid,
        in_specs=[
            pl.BlockSpec((1, _NC, _K, _D), lambda i, j: (i, j, 0, 0)),
            pl.BlockSpec((1, _NC, _D), lambda i, j: (i, j, 0)),
            pl.BlockSpec((_D, cout), lambda i, j: (0, 0)),
            pl.BlockSpec((8, cout), lambda i, j: (0, 0)),
            pl.BlockSpec((cout, cout), lambda i, j: (0, 0)),
            pl.BlockSpec((8, cout), lambda i, j: (0, 0)),
        ],
        out_specs=pl.BlockSpec((1, cout, _NC), lambda i, j: (i, 0, j)),
        out_shape=jax.ShapeDtypeStruct((b, cout, n), jnp.float32),
    )(gathered4, centers3, w1t, b1, w2t, b2)


def kernel(xyz, conv1_w, conv1_b, conv2_w, conv2_b, bn_w, bn_b):
    b, _, n = xyz.shape
    cout = conv1_w.shape[0]

    idx = _knn_indices(xyz)

    pts = jnp.transpose(xyz, (0, 2, 1))
    table = jnp.concatenate(
        [pts, jnp.zeros((b, n, _D - 3), jnp.float32)], axis=-1
    ).reshape(b * n, _D)
    idx2d = idx.reshape(-1, _CH)
    gathered = _gather_rows(table, idx2d)
    gathered4 = gathered.reshape(b, n, _K, _D)
    centers3 = table.reshape(b, n, _D)

    smat, vmat = _moments(gathered4, centers3)
    cnt = jnp.float32(b * n * _K)
    mu = vmat[0] / cnt
    sig = smat / cnt - jnp.outer(mu, mu)
    w1p = jnp.concatenate(
        [conv1_w, jnp.zeros((cout, _D - 3), jnp.float32)], axis=1
    )
    mean_c = w1p @ mu + conv1_b
    var_c = jnp.sum((w1p @ sig) * w1p, axis=1)
    scale = bn_w * lax.rsqrt(var_c + 1e-5)
    w1f = w1p * scale[:, None]
    b1f = (conv1_b - mean_c) * scale + bn_b

    b1m = jnp.broadcast_to(b1f[None, :], (8, cout))
    b2m = jnp.broadcast_to(conv2_b[None, :], (8, cout))
    return _mlp(gathered4, centers3, w1f.T, b1m, conv2_w.T, b2m)


# speedup vs baseline: 22.2731x; 1.7031x over previous
import functools

import jax
import jax.numpy as jnp
from jax import lax
from jax.experimental import pallas as pl
from jax.experimental.pallas import tpu as pltpu
from jax.experimental.pallas import tpu_sc as plsc

_K = 16
_D = 16
_TR = 256
_NM = 512
_NC = 512
_CH = 128


_LANES = 128
_T_CAND = 4


def _tree_min(parts):
    while len(parts) > 1:
        nxt = [jnp.minimum(parts[i], parts[i + 1])
               for i in range(0, len(parts) - 1, 2)]
        if len(parts) % 2:
            nxt.append(parts[-1])
        parts = nxt
    return parts[0]


def _seg_min(a, c, l):
    return _tree_min([a[:, i * l:(i + 1) * l] for i in range(c)])


def _knn_body(xyz_full_ref, xyz_tile_ref, idx_ref):
    b = pl.program_id(0)
    n = xyz_full_ref.shape[2]
    tr = xyz_tile_ref.shape[2]
    c = n // _LANES
    x = xyz_full_ref[0]
    xt = xyz_tile_ref[0]
    sq = jnp.sum(x * x, axis=0, keepdims=True)
    sqt = jnp.sum(xt * xt, axis=0)[:, None]
    cross = lax.dot_general(xt, x, (((0,), (0,)), ((), ())),
                            preferred_element_type=jnp.float32)
    d = (sqt + sq) - 2.0 * cross
    iota = lax.broadcasted_iota(jnp.int32, (tr, n), 1)
    inf = jnp.float32(jnp.inf)
    bigi = jnp.int32(n)

    cvals, cidxs = [], []
    for _ in range(_T_CAND):
        m = _seg_min(d, c, _LANES)
        mb = jnp.concatenate([m] * c, axis=1)
        eq = d == mb
        pos = _seg_min(jnp.where(eq, iota, bigi), c, _LANES)
        cvals.append(m)
        cidxs.append(pos)
        d = jnp.where(eq, inf, d)

    v = jnp.concatenate(cvals, axis=1)
    pi = jnp.concatenate(cidxs, axis=1)

    cols = []
    for k in range(_K + 1):
        m = jnp.min(v, axis=1, keepdims=True)
        eq = v == m
        sel = jnp.min(jnp.where(eq, pi, bigi), axis=1, keepdims=True)
        if k > 0:
            cols.append(sel)
        v = jnp.where(eq & (pi == sel), inf, v)
    idxs = jnp.concatenate(cols, axis=1)
    idx_ref[0] = idxs + b * n


def _knn_indices(xyz):
    b, _, n = xyz.shape
    grid = (b, n // _TR)
    return pl.pallas_call(
        _knn_body,
        grid=grid,
        in_specs=[
            pl.BlockSpec((1, 3, n), lambda i, j: (i, 0, 0)),
            pl.BlockSpec((1, 3, _TR), lambda i, j: (i, 0, j)),
        ],
        out_specs=pl.BlockSpec((1, _TR, _K), lambda i, j: (i, j, 0)),
        out_shape=jax.ShapeDtypeStruct((b, n, _K), jnp.int32),
    )(xyz, xyz)


def _gather_rows(table, idx2d):
    rows_total = idx2d.shape[0] * idx2d.shape[1]
    info = plsc.get_sparse_core_info()
    nw = info.num_cores * info.num_subcores
    per_w = rows_total // nw
    n_ch = per_w // _CH

    mesh = plsc.VectorSubcoreMesh(core_axis_name="c", subcore_axis_name="s")

    @functools.partial(
        pl.kernel,
        mesh=mesh,
        compiler_params=pltpu.CompilerParams(use_tc_tiling_on_sc=False),
        out_type=jax.ShapeDtypeStruct((rows_total, _D), jnp.float32),
        scratch_types=[
            pltpu.VMEM((n_ch, _CH), jnp.int32),
            pltpu.VMEM((_CH, _D), jnp.float32),
            pltpu.SemaphoreType.DMA,
        ],
    )
    def _gather_kernel(table_hbm, idx_hbm, out_hbm, idx_v, rows_v, sem):
        wid = lax.axis_index("s") * info.num_cores + lax.axis_index("c")
        base = wid * per_w
        pltpu.sync_copy(idx_hbm.at[pl.ds(wid * n_ch, n_ch)], idx_v)

        def body(j, carry):
            pltpu.async_copy(table_hbm.at[idx_v.at[j]], rows_v, sem).wait()
            pltpu.sync_copy(rows_v, out_hbm.at[pl.ds(base + j * _CH, _CH)])
            return carry

        lax.fori_loop(0, n_ch, body, 0)

    return _gather_kernel(table, idx2d)


def _moments_body(g_ref, c_ref, s_ref, v_ref):
    first = (pl.program_id(0) == 0) & (pl.program_id(1) == 0)
    g = g_ref[0]
    c = c_ref[0]
    diffs = g - c[:, None, :]
    xm = diffs.reshape(g.shape[0] * _K, _D)
    smat = lax.dot_general(xm, xm, (((0,), (0,)), ((), ())),
                           preferred_element_type=jnp.float32)
    vrow = jnp.sum(xm, axis=0, keepdims=True)

    @pl.when(first)
    def _():
        s_ref[...] = jnp.zeros_like(s_ref)
        v_ref[...] = jnp.zeros_like(v_ref)

    s_ref[...] += smat
    v_ref[0:1, :] += vrow


def _moments(gathered4, centers3):
    b, n = centers3.shape[0], centers3.shape[1]
    grid = (b, n // _NM)
    return pl.pallas_call(
        _moments_body,
        grid=grid,
        in_specs=[
            pl.BlockSpec((1, _NM, _K, _D), lambda i, j: (i, j, 0, 0)),
            pl.BlockSpec((1, _NM, _D), lambda i, j: (i, j, 0)),
        ],
        out_specs=[
            pl.BlockSpec((_D, _D), lambda i, j: (0, 0)),
            pl.BlockSpec((8, _D), lambda i, j: (0, 0)),
        ],
        out_shape=[
            jax.ShapeDtypeStruct((_D, _D), jnp.float32),
            jax.ShapeDtypeStruct((8, _D), jnp.float32),
        ],
    )(gathered4, centers3)


def _mlp_body(g_ref, c_ref, w1_ref, b1_ref, w2_ref, b2_ref, o_ref):
    g = g_ref[0]
    c = c_ref[0]
    diffs = g - c[:, None, :]
    xm = diffs.reshape(g.shape[0] * _K, _D)
    h = jnp.dot(xm, w1_ref[...], preferred_element_type=jnp.float32)
    h = jnp.maximum(h + b1_ref[0:1, :], 0.0)
    z = jnp.dot(h, w2_ref[...], preferred_element_type=jnp.float32)
    z = z + b2_ref[0:1, :]
    z3 = z.reshape(g.shape[0], _K, z.shape[1])
    o = jnp.max(z3, axis=1)
    o_ref[0] = jnp.transpose(o)


def _mlp(gathered4, centers3, w1t, b1, w2t, b2):
    b, n = centers3.shape[0], centers3.shape[1]
    cout = w2t.shape[1]
    grid = (b, n // _NC)
    return pl.pallas_call(
        _mlp_body,
        grid=grid,
        in_specs=[
            pl.BlockSpec((1, _NC, _K, _D), lambda i, j: (i, j, 0, 0)),
            pl.BlockSpec((1, _NC, _D), lambda i, j: (i, j, 0)),
            pl.BlockSpec((_D, cout), lambda i, j: (0, 0)),
            pl.BlockSpec((8, cout), lambda i, j: (0, 0)),
            pl.BlockSpec((cout, cout), lambda i, j: (0, 0)),
            pl.BlockSpec((8, cout), lambda i, j: (0, 0)),
        ],
        out_specs=pl.BlockSpec((1, cout, _NC), lambda i, j: (i, 0, j)),
        out_shape=jax.ShapeDtypeStruct((b, cout, n), jnp.float32),
    )(gathered4, centers3, w1t, b1, w2t, b2)


def kernel(xyz, conv1_w, conv1_b, conv2_w, conv2_b, bn_w, bn_b):
    b, _, n = xyz.shape
    cout = conv1_w.shape[0]

    idx = _knn_indices(xyz)

    pts = jnp.transpose(xyz, (0, 2, 1))
    table = jnp.concatenate(
        [pts, jnp.zeros((b, n, _D - 3), jnp.float32)], axis=-1
    ).reshape(b * n, _D)
    idx2d = idx.reshape(-1, _CH)
    gathered = _gather_rows(table, idx2d)
    gathered4 = gathered.reshape(b, n, _K, _D)
    centers3 = table.reshape(b, n, _D)

    smat, vmat = _moments(gathered4, centers3)
    cnt = jnp.float32(b * n * _K)
    mu = vmat[0] / cnt
    sig = smat / cnt - jnp.outer(mu, mu)
    w1p = jnp.concatenate(
        [conv1_w, jnp.zeros((cout, _D - 3), jnp.float32)], axis=1
    )
    mean_c = w1p @ mu + conv1_b
    var_c = jnp.sum((w1p @ sig) * w1p, axis=1)
    scale = bn_w * lax.rsqrt(var_c + 1e-5)
    w1f = w1p * scale[:, None]
    b1f = (conv1_b - mean_c) * scale + bn_b

    b1m = jnp.broadcast_to(b1f[None, :], (8, cout))
    b2m = jnp.broadcast_to(conv2_b[None, :], (8, cout))
    return _mlp(gathered4, centers3, w1f.T, b1m, conv2_w.T, b2m)


# speedup vs baseline: 29.7478x; 1.3356x over previous
import functools

import jax
import jax.numpy as jnp
from jax import lax
from jax.experimental import pallas as pl
from jax.experimental.pallas import tpu as pltpu
from jax.experimental.pallas import tpu_sc as plsc

_K = 16
_D = 16
_TR = 256
_NM = 512
_NC = 512
_CH = 128


_LANES = 128
_T_CAND = 4


def _tree_min(parts):
    while len(parts) > 1:
        nxt = [jnp.minimum(parts[i], parts[i + 1])
               for i in range(0, len(parts) - 1, 2)]
        if len(parts) % 2:
            nxt.append(parts[-1])
        parts = nxt
    return parts[0]


def _seg_min(a, c, l):
    return _tree_min([a[:, i * l:(i + 1) * l] for i in range(c)])


def _knn_body(xyz_full_ref, xyz_tile_ref, idx_ref):
    b = pl.program_id(0)
    n = xyz_full_ref.shape[2]
    tr = xyz_tile_ref.shape[2]
    c = n // _LANES
    x = xyz_full_ref[0]
    xt = xyz_tile_ref[0]
    sq = jnp.sum(x * x, axis=0, keepdims=True)
    sqt = jnp.sum(xt * xt, axis=0)[:, None]
    cross = lax.dot_general(xt * jnp.float32(-2.0), x,
                            (((0,), (0,)), ((), ())),
                            preferred_element_type=jnp.float32)
    d = (sqt + sq) + cross
    inf = jnp.float32(jnp.inf)

    d = jnp.abs(jnp.maximum(d, 0.0))
    chunk_row = lax.broadcasted_iota(jnp.int32, (1, n), 1) >> 7
    keys = lax.bitcast_convert_type(
        (lax.bitcast_convert_type(d, jnp.int32) & jnp.int32(~31)) | chunk_row,
        jnp.float32)

    cvals = []
    for _ in range(_T_CAND):
        m = _seg_min(keys, c, _LANES)
        cvals.append(m)
        mb = jnp.concatenate([m] * c, axis=1)
        keys = jnp.where(keys == mb, inf, keys)

    v = jnp.concatenate(cvals, axis=1)
    nc = _T_CAND * _LANES
    iota_f = lax.broadcasted_iota(jnp.int32, (tr, nc), 1).astype(jnp.float32)
    bigf = jnp.float32(nc)

    sels = []
    for k in range(_K + 1):
        m = jnp.min(v, axis=1, keepdims=True)
        eq = v == m
        if k == 0:
            v = jnp.where(eq, inf, v)
            continue
        slot = jnp.min(jnp.where(eq, iota_f, bigf), axis=1, keepdims=True)
        sels.append((m, slot))
        if k < _K:
            v = jnp.where(iota_f == slot, inf, v)
    cols = []
    for m, slot in sels:
        chunk = lax.bitcast_convert_type(m, jnp.int32) & jnp.int32(31)
        lane = slot.astype(jnp.int32) & jnp.int32(_LANES - 1)
        cols.append((chunk << 7) | lane)
    idxs = jnp.concatenate(cols, axis=1)
    idx_ref[0] = idxs + b * n


def _knn_indices(xyz):
    b, _, n = xyz.shape
    grid = (b, n // _TR)
    return pl.pallas_call(
        _knn_body,
        grid=grid,
        in_specs=[
            pl.BlockSpec((1, 3, n), lambda i, j: (i, 0, 0)),
            pl.BlockSpec((1, 3, _TR), lambda i, j: (i, 0, j)),
        ],
        out_specs=pl.BlockSpec((1, _TR, _K), lambda i, j: (i, j, 0)),
        out_shape=jax.ShapeDtypeStruct((b, n, _K), jnp.int32),
    )(xyz, xyz)


def _gather_rows(table, idx2d):
    rows_total = idx2d.shape[0] * idx2d.shape[1]
    info = plsc.get_sparse_core_info()
    nw = info.num_cores * info.num_subcores
    per_w = rows_total // nw
    n_ch = per_w // _CH

    mesh = plsc.VectorSubcoreMesh(core_axis_name="c", subcore_axis_name="s")

    @functools.partial(
        pl.kernel,
        mesh=mesh,
        compiler_params=pltpu.CompilerParams(use_tc_tiling_on_sc=False),
        out_type=jax.ShapeDtypeStruct((rows_total, _D), jnp.float32),
        scratch_types=[
            pltpu.VMEM((n_ch, _CH), jnp.int32),
            pltpu.VMEM((_CH, _D), jnp.float32),
            pltpu.SemaphoreType.DMA,
        ],
    )
    def _gather_kernel(table_hbm, idx_hbm, out_hbm, idx_v, rows_v, sem):
        wid = lax.axis_index("s") * info.num_cores + lax.axis_index("c")
        base = wid * per_w
        pltpu.sync_copy(idx_hbm.at[pl.ds(wid * n_ch, n_ch)], idx_v)

        def body(j, carry):
            pltpu.async_copy(table_hbm.at[idx_v.at[j]], rows_v, sem).wait()
            pltpu.sync_copy(rows_v, out_hbm.at[pl.ds(base + j * _CH, _CH)])
            return carry

        lax.fori_loop(0, n_ch, body, 0)

    return _gather_kernel(table, idx2d)


def _moments_body(g_ref, c_ref, s_ref, v_ref):
    first = (pl.program_id(0) == 0) & (pl.program_id(1) == 0)
    g = g_ref[0]
    c = c_ref[0]
    diffs = g - c[:, None, :]
    xm = diffs.reshape(g.shape[0] * _K, _D)
    smat = lax.dot_general(xm, xm, (((0,), (0,)), ((), ())),
                           preferred_element_type=jnp.float32)
    vrow = jnp.sum(xm, axis=0, keepdims=True)

    @pl.when(first)
    def _():
        s_ref[...] = jnp.zeros_like(s_ref)
        v_ref[...] = jnp.zeros_like(v_ref)

    s_ref[...] += smat
    v_ref[0:1, :] += vrow


def _moments(gathered4, centers3):
    b, n = centers3.shape[0], centers3.shape[1]
    grid = (b, n // _NM)
    return pl.pallas_call(
        _moments_body,
        grid=grid,
        in_specs=[
            pl.BlockSpec((1, _NM, _K, _D), lambda i, j: (i, j, 0, 0)),
            pl.BlockSpec((1, _NM, _D), lambda i, j: (i, j, 0)),
        ],
        out_specs=[
            pl.BlockSpec((_D, _D), lambda i, j: (0, 0)),
            pl.BlockSpec((8, _D), lambda i, j: (0, 0)),
        ],
        out_shape=[
            jax.ShapeDtypeStruct((_D, _D), jnp.float32),
            jax.ShapeDtypeStruct((8, _D), jnp.float32),
        ],
    )(gathered4, centers3)


def _mlp_body(g_ref, c_ref, w1_ref, b1_ref, w2_ref, b2_ref, o_ref):
    g = g_ref[0]
    c = c_ref[0]
    diffs = g - c[:, None, :]
    xm = diffs.reshape(g.shape[0] * _K, _D)
    h = jnp.dot(xm, w1_ref[...], preferred_element_type=jnp.float32)
    h = jnp.maximum(h + b1_ref[0:1, :], 0.0)
    z = jnp.dot(h, w2_ref[...], preferred_element_type=jnp.float32)
    z = z + b2_ref[0:1, :]
    z3 = z.reshape(g.shape[0], _K, z.shape[1])
    o = jnp.max(z3, axis=1)
    o_ref[0] = jnp.transpose(o)


def _mlp(gathered4, centers3, w1t, b1, w2t, b2):
    b, n = centers3.shape[0], centers3.shape[1]
    cout = w2t.shape[1]
    grid = (b, n // _NC)
    return pl.pallas_call(
        _mlp_body,
        grid=grid,
        in_specs=[
            pl.BlockSpec((1, _NC, _K, _D), lambda i, j: (i, j, 0, 0)),
            pl.BlockSpec((1, _NC, _D), lambda i, j: (i, j, 0)),
            pl.BlockSpec((_D, cout), lambda i, j: (0, 0)),
            pl.BlockSpec((8, cout), lambda i, j: (0, 0)),
            pl.BlockSpec((cout, cout), lambda i, j: (0, 0)),
            pl.BlockSpec((8, cout), lambda i, j: (0, 0)),
        ],
        out_specs=pl.BlockSpec((1, cout, _NC), lambda i, j: (i, 0, j)),
        out_shape=jax.ShapeDtypeStruct((b, cout, n), jnp.float32),
    )(gathered4, centers3, w1t, b1, w2t, b2)


def kernel(xyz, conv1_w, conv1_b, conv2_w, conv2_b, bn_w, bn_b):
    b, _, n = xyz.shape
    cout = conv1_w.shape[0]

    idx = _knn_indices(xyz)

    pts = jnp.transpose(xyz, (0, 2, 1))
    table = jnp.concatenate(
        [pts, jnp.zeros((b, n, _D - 3), jnp.float32)], axis=-1
    ).reshape(b * n, _D)
    idx2d = idx.reshape(-1, _CH)
    gathered = _gather_rows(table, idx2d)
    gathered4 = gathered.reshape(b, n, _K, _D)
    centers3 = table.reshape(b, n, _D)

    smat, vmat = _moments(gathered4, centers3)
    cnt = jnp.float32(b * n * _K)
    mu = vmat[0] / cnt
    sig = smat / cnt - jnp.outer(mu, mu)
    w1p = jnp.concatenate(
        [conv1_w, jnp.zeros((cout, _D - 3), jnp.float32)], axis=1
    )
    mean_c = w1p @ mu + conv1_b
    var_c = jnp.sum((w1p @ sig) * w1p, axis=1)
    scale = bn_w * lax.rsqrt(var_c + 1e-5)
    w1f = w1p * scale[:, None]
    b1f = (conv1_b - mean_c) * scale + bn_b

    b1m = jnp.broadcast_to(b1f[None, :], (8, cout))
    b2m = jnp.broadcast_to(conv2_b[None, :], (8, cout))
    return _mlp(gathered4, centers3, w1f.T, b1m, conv2_w.T, b2m)
